# baseline (speedup 1.0000x reference)
import jax
import jax.numpy as jnp
from jax import lax
from jax.experimental import pallas as pl
from jax.experimental.pallas import tpu as pltpu

K = 20
N = 1024
NEG = -3.0e38


def _leaky(z):
    return jnp.where(z > 0, z, 0.2 * z)


def _dgcnn_body(x_ref,
                ec1_W, ec1_b, ec2_W, ec2_b, ec3_W, ec3_b, ec4_W, ec4_b,
                agg_W, agg_b,
                xfeat_ref,
                D_ref, F_ref):
    X = x_ref[0]
    col = 0
    for (C, Cout, li) in ((8, 64, 0), (64, 64, 1), (64, 128, 2), (128, 256, 3)):
        W = (ec1_W, ec2_W, ec3_W, ec4_W)[li][...]
        bvec = (ec1_b, ec2_b, ec3_b, ec4_b)[li][...]
        Xhib = X.astype(jnp.bfloat16)
        r1 = X - Xhib.astype(jnp.float32)
        Xmidb = r1.astype(jnp.bfloat16)
        Xlob = (r1 - Xmidb.astype(jnp.float32)).astype(jnp.bfloat16)
        sq = jnp.sum(X * X, axis=1)
        G = lax.dot_general(X, X, (((1,), (1,)), ((), ())),
                            preferred_element_type=jnp.float32)
        D_ref[...] = 2.0 * G - sq[:, None] - sq[None, :]

        jj = lax.broadcasted_iota(jnp.int32, (N, N), 1)

        def body(t, M):
            D = D_ref[...]
            m = jnp.max(D, axis=1)
            eq = D == m[:, None]
            idx = jnp.min(jnp.where(eq, jj, N + 1), axis=1)
            H = jj == idx[:, None]
            D_ref[...] = jnp.where(H, NEG, D)
            Hb = H.astype(jnp.bfloat16)
            S = ((jnp.dot(Hb, Xhib, preferred_element_type=jnp.float32)
                  + jnp.dot(Hb, Xmidb, preferred_element_type=jnp.float32))
                 + jnp.dot(Hb, Xlob, preferred_element_type=jnp.float32))
            e = jnp.concatenate([S - X, X], axis=1)
            h = jnp.dot(e, W, preferred_element_type=jnp.float32) + bvec
            return jnp.maximum(M, h)

        M = lax.fori_loop(0, K, body, jnp.full((N, Cout), NEG, jnp.float32))
        Xn = _leaky(M)
        F_ref[:, col:col + Cout] = Xn
        col += Cout
        X = Xn

    f = _leaky(jnp.dot(F_ref[...], agg_W[...],
                       preferred_element_type=jnp.float32) + agg_b[...])
    xfeat_ref[0, 0] = jnp.max(f, axis=0)


def _head_body(xfeat_ref, metrics_ref,
               tab_W0, tab_b0,
               att_W0, att_b0, feat_W0, feat_b0,
               att_W1, att_b1, feat_W1, feat_b1,
               att_W2, att_b2, feat_W2, feat_b2,
               att_W3, att_b3, feat_W3, feat_b3,
               att_W4, att_b4, feat_W4, feat_b4,
               h1_W, h1_b, h2_W, h2_b, h3_W, h3_b, h4_W, h4_b,
               amax_ref, probs_ref, logits_ref, mfeats_ref):
    metrics = metrics_ref[...]
    n_d = 64
    gamma = 1.5

    def mm(a, w, b):
        return jnp.dot(a, w[...], preferred_element_type=jnp.float32) + b[...]

    prior = jnp.ones_like(metrics)
    feat0 = jnp.maximum(mm(metrics, tab_W0, tab_b0), 0.0)
    a = feat0[:, n_d:]
    atts = ((att_W0, att_b0, feat_W0, feat_b0),
            (att_W1, att_b1, feat_W1, feat_b1),
            (att_W2, att_b2, feat_W2, feat_b2),
            (att_W3, att_b3, feat_W3, feat_b3),
            (att_W4, att_b4, feat_W4, feat_b4))
    for i, (aW, ab, fW, fb) in enumerate(atts):
        mask_logits = mm(a, aW, ab) * prior
        z = mask_logits - jnp.max(mask_logits, axis=-1, keepdims=True)
        ez = jnp.exp(z)
        mask = ez / jnp.sum(ez, axis=-1, keepdims=True)
        prior = prior * (gamma - mask)
        masked = metrics * mask
        feat = jnp.maximum(mm(masked, fW, fb), 0.0)
        mfeats_ref[:, i * n_d:(i + 1) * n_d] = feat[:, :n_d]
        a = feat[:, n_d:]

    feats = jnp.concatenate([xfeat_ref[...], mfeats_ref[...]], axis=1)
    h = jnp.maximum(mm(feats, h1_W, h1_b), 0.0)
    h = jnp.maximum(mm(h, h2_W, h2_b), 0.0)
    h = jnp.maximum(mm(h, h3_W, h3_b), 0.0)
    logits = mm(h, h4_W, h4_b)
    logits_ref[...] = logits
    z = logits - jnp.max(logits, axis=-1, keepdims=True)
    ez = jnp.exp(z)
    probs = ez / jnp.sum(ez, axis=-1, keepdims=True)
    probs_ref[...] = probs
    nc = probs.shape[1]
    ii = lax.broadcasted_iota(jnp.int32, probs.shape, 1)
    pm = jnp.max(probs, axis=1, keepdims=True)
    amax_ref[...] = jnp.min(jnp.where(probs == pm, ii, nc + 1), axis=1,
                            keepdims=True)


def kernel(x, metrics, params):
    p = params
    B = x.shape[0]

    def full(s):
        return pl.BlockSpec(s, lambda *_: (0,) * len(s))

    wspecs = []
    wvals = []
    for name in ('ec1', 'ec2', 'ec3', 'ec4'):
        W = p[name + '_W']
        bv = p[name + '_b'].reshape(1, -1)
        wvals += [W, bv]
        wspecs += [full(W.shape), full(bv.shape)]
    aggW = p['agg_W']
    aggb = p['agg_b'].reshape(1, -1)
    wvals += [aggW, aggb]
    wspecs += [full(aggW.shape), full(aggb.shape)]

    x_feats = pl.pallas_call(
        _dgcnn_body,
        grid=(B,),
        in_specs=[pl.BlockSpec((1, N, x.shape[2]), lambda b: (b, 0, 0))] + wspecs,
        out_specs=pl.BlockSpec((1, 1, 128), lambda b: (b, 0, 0)),
        out_shape=jax.ShapeDtypeStruct((B, 1, 128), jnp.float32),
        scratch_shapes=[
            pltpu.VMEM((N, N), jnp.float32),
            pltpu.VMEM((N, 512), jnp.float32),
        ],
    )(x, *wvals)
    x_feats = x_feats.reshape(B, 128)

    hvals = [x_feats, metrics, p['tab_W0'], p['tab_b0'].reshape(1, -1)]
    for i in range(5):
        hvals += [p['tab_att_W%d' % i], p['tab_att_b%d' % i].reshape(1, -1),
                  p['tab_feat_W%d' % i], p['tab_feat_b%d' % i].reshape(1, -1)]
    for nm in ('h1', 'h2', 'h3', 'h4'):
        hvals += [p[nm + '_W'], p[nm + '_b'].reshape(1, -1)]
    hspecs = [full(v.shape) for v in hvals]

    amax, probs, logits, mfeats = pl.pallas_call(
        _head_body,
        in_specs=hspecs,
        out_specs=[full((B, 1)), full((B, 50)), full((B, 50)), full((B, 320))],
        out_shape=[
            jax.ShapeDtypeStruct((B, 1), jnp.int32),
            jax.ShapeDtypeStruct((B, 50), jnp.float32),
            jax.ShapeDtypeStruct((B, 50), jnp.float32),
            jax.ShapeDtypeStruct((B, 320), jnp.float32),
        ],
    )(*hvals)

    return (amax.reshape(B), probs, logits, x_feats, mfeats)


# speedup vs baseline: 1.3589x; 1.3589x over previous
import functools
import jax
import jax.numpy as jnp
from jax import lax
from jax.experimental import pallas as pl
from jax.experimental.pallas import tpu as pltpu
from jax.experimental.pallas import tpu_sc as plsc

K = 20
N = 1024
NEG = -3.0e38
NW = 32
CH = 128


def _leaky(z):
    return jnp.where(z > 0, z, 0.2 * z)


def _dist(X, D_ref):
    sq = jnp.sum(X * X, axis=1)
    G = lax.dot_general(X, X, (((1,), (1,)), ((), ())),
                        preferred_element_type=jnp.float32)
    D_ref[...] = 2.0 * G - sq[:, None] - sq[None, :]


def _topk_idx(D_ref, boff):
    jj = lax.broadcasted_iota(jnp.int32, (N, N), 1)
    rr = lax.broadcasted_iota(jnp.int32, (K, N), 0)

    def body(t, Imat):
        D = D_ref[...]
        m = jnp.max(D, axis=1)
        eq = D == m[:, None]
        idx = jnp.min(jnp.where(eq, jj, N + 1), axis=1)
        H = jj == idx[:, None]
        D_ref[...] = jnp.where(H, NEG, D)
        return jnp.where(rr == t, (idx + boff)[None, :], Imat)

    return lax.fori_loop(0, K, body, jnp.zeros((K, N), jnp.int32))


def _sel1_body(x_ref, idx_ref, D_ref):
    b = pl.program_id(0)
    _dist(x_ref[0], D_ref)
    idx_ref[0] = _topk_idx(D_ref, b * N)


def _conv(X, nbrs_ref, W, bvec, C, Cout):
    M = jnp.full((N, Cout), NEG, jnp.float32)
    for t in range(K):
        Sn = nbrs_ref[0, t][:, :C]
        e = jnp.concatenate([Sn - X, X], axis=1)
        h = jnp.dot(e, W, preferred_element_type=jnp.float32) + bvec
        M = jnp.maximum(M, h)
    return _leaky(M)


def _convsel_body(C, Cout, xprev_ref, nbrs_ref, W_ref, b_ref,
                  xout_ref, idx_ref, D_ref):
    b = pl.program_id(0)
    X = xprev_ref[0][:, :C]
    Xn = _conv(X, nbrs_ref, W_ref[...], b_ref[...], C, Cout)
    if Cout < 128:
        xout_ref[0] = jnp.concatenate(
            [Xn, jnp.zeros((N, 128 - Cout), jnp.float32)], axis=1)
    else:
        xout_ref[0] = Xn
    _dist(Xn, D_ref)
    idx_ref[0] = _topk_idx(D_ref, b * N)


def _fin_body(x3_ref, nbrs_ref, W_ref, b_ref, x1_ref, x2_ref,
              agg_W, agg_b, xfeat_ref):
    X3 = x3_ref[0]
    X4 = _conv(X3, nbrs_ref, W_ref[...], b_ref[...], 128, 256)
    F = jnp.concatenate([x1_ref[0][:, :64], x2_ref[0][:, :64], X3, X4],
                        axis=1)
    f = _leaky(jnp.dot(F, agg_W[...], preferred_element_type=jnp.float32)
               + agg_b[...])
    xfeat_ref[0, 0] = jnp.max(f, axis=0)


def _sc_gather(table, idx):
    M = idx.shape[0]
    Cp = table.shape[1]
    per_w = M // NW
    nch = per_w // CH
    mesh = plsc.VectorSubcoreMesh(core_axis_name="c", subcore_axis_name="s")

    @functools.partial(
        pl.kernel, mesh=mesh,
        out_type=jax.ShapeDtypeStruct((M, Cp), jnp.float32),
        scratch_types=[
            pltpu.VMEM((CH,), jnp.int32),
            pltpu.VMEM((CH, Cp), jnp.float32),
            pltpu.SemaphoreType.DMA,
        ],
    )
    def k(table_hbm, idx_hbm, out_hbm, idxc_v, rows_v, sem):
        wid = lax.axis_index("s") * 2 + lax.axis_index("c")
        base = wid * per_w

        def chunk(i, _):
            off = base + i * CH
            pltpu.sync_copy(idx_hbm.at[pl.ds(off, CH)], idxc_v)
            pltpu.async_copy(table_hbm.at[idxc_v], rows_v, sem).wait()
            pltpu.sync_copy(rows_v, out_hbm.at[pl.ds(off, CH)])
            return 0

        lax.fori_loop(0, nch, chunk, 0)

    return k(table, idx)


def _head_body(xfeat_ref, metrics_ref,
               tab_W0, tab_b0,
               att_W0, att_b0, feat_W0, feat_b0,
               att_W1, att_b1, feat_W1, feat_b1,
               att_W2, att_b2, feat_W2, feat_b2,
               att_W3, att_b3, feat_W3, feat_b3,
               att_W4, att_b4, feat_W4, feat_b4,
               h1_W, h1_b, h2_W, h2_b, h3_W, h3_b, h4_W, h4_b,
               amax_ref, probs_ref, logits_ref, mfeats_ref):
    metrics = metrics_ref[...]
    n_d = 64
    gamma = 1.5

    def mm(a, w, b):
        return jnp.dot(a, w[...], preferred_element_type=jnp.float32) + b[...]

    prior = jnp.ones_like(metrics)
    feat0 = jnp.maximum(mm(metrics, tab_W0, tab_b0), 0.0)
    a = feat0[:, n_d:]
    atts = ((att_W0, att_b0, feat_W0, feat_b0),
            (att_W1, att_b1, feat_W1, feat_b1),
            (att_W2, att_b2, feat_W2, feat_b2),
            (att_W3, att_b3, feat_W3, feat_b3),
            (att_W4, att_b4, feat_W4, feat_b4))
    for i, (aW, ab, fW, fb) in enumerate(atts):
        mask_logits = mm(a, aW, ab) * prior
        z = mask_logits - jnp.max(mask_logits, axis=-1, keepdims=True)
        ez = jnp.exp(z)
        mask = ez / jnp.sum(ez, axis=-1, keepdims=True)
        prior = prior * (gamma - mask)
        masked = metrics * mask
        feat = jnp.maximum(mm(masked, fW, fb), 0.0)
        mfeats_ref[:, i * n_d:(i + 1) * n_d] = feat[:, :n_d]
        a = feat[:, n_d:]

    feats = jnp.concatenate([xfeat_ref[...], mfeats_ref[...]], axis=1)
    h = jnp.maximum(mm(feats, h1_W, h1_b), 0.0)
    h = jnp.maximum(mm(h, h2_W, h2_b), 0.0)
    h = jnp.maximum(mm(h, h3_W, h3_b), 0.0)
    logits = mm(h, h4_W, h4_b)
    logits_ref[...] = logits
    z = logits - jnp.max(logits, axis=-1, keepdims=True)
    ez = jnp.exp(z)
    probs = ez / jnp.sum(ez, axis=-1, keepdims=True)
    probs_ref[...] = probs
    nc = probs.shape[1]
    ii = lax.broadcasted_iota(jnp.int32, probs.shape, 1)
    pm = jnp.max(probs, axis=1, keepdims=True)
    amax_ref[...] = jnp.min(jnp.where(probs == pm, ii, nc + 1), axis=1,
                            keepdims=True)


def _full(s):
    return pl.BlockSpec(s, lambda *_: (0,) * len(s))


def _sel1(x):
    B = x.shape[0]
    return pl.pallas_call(
        _sel1_body,
        grid=(B,),
        in_specs=[pl.BlockSpec((1, N, x.shape[2]), lambda b: (b, 0, 0))],
        out_specs=pl.BlockSpec((1, K, N), lambda b: (b, 0, 0)),
        out_shape=jax.ShapeDtypeStruct((B, K, N), jnp.int32),
        scratch_shapes=[pltpu.VMEM((N, N), jnp.float32)],
    )(x)


def _convsel(xprev, nbrs, W, bvec, C, Cout):
    B = xprev.shape[0]
    Cp = nbrs.shape[-1]
    return pl.pallas_call(
        functools.partial(_convsel_body, C, Cout),
        grid=(B,),
        in_specs=[
            pl.BlockSpec((1, N, xprev.shape[2]), lambda b: (b, 0, 0)),
            pl.BlockSpec((1, K, N, Cp), lambda b: (b, 0, 0, 0)),
            _full(W.shape), _full(bvec.shape),
        ],
        out_specs=[
            pl.BlockSpec((1, N, 128), lambda b: (b, 0, 0)),
            pl.BlockSpec((1, K, N), lambda b: (b, 0, 0)),
        ],
        out_shape=[
            jax.ShapeDtypeStruct((B, N, 128), jnp.float32),
            jax.ShapeDtypeStruct((B, K, N), jnp.int32),
        ],
        scratch_shapes=[pltpu.VMEM((N, N), jnp.float32)],
    )(xprev, nbrs, W, bvec)


def _fin(x3, nbrs, W, bvec, x1, x2, aggW, aggb):
    B = x3.shape[0]
    Cp = nbrs.shape[-1]
    return pl.pallas_call(
        _fin_body,
        grid=(B,),
        in_specs=[
            pl.BlockSpec((1, N, 128), lambda b: (b, 0, 0)),
            pl.BlockSpec((1, K, N, Cp), lambda b: (b, 0, 0, 0)),
            _full(W.shape), _full(bvec.shape),
            pl.BlockSpec((1, N, 128), lambda b: (b, 0, 0)),
            pl.BlockSpec((1, N, 128), lambda b: (b, 0, 0)),
            _full(aggW.shape), _full(aggb.shape),
        ],
        out_specs=pl.BlockSpec((1, 1, 128), lambda b: (b, 0, 0)),
        out_shape=jax.ShapeDtypeStruct((B, 1, 128), jnp.float32),
    )(x3, nbrs, W, bvec, x1, x2, aggW, aggb)


def kernel(x, metrics, params):
    p = params
    B = x.shape[0]

    ec = {nm: (p[nm + '_W'], p[nm + '_b'].reshape(1, -1))
          for nm in ('ec1', 'ec2', 'ec3', 'ec4')}
    aggW = p['agg_W']
    aggb = p['agg_b'].reshape(1, -1)

    xpad = jnp.pad(x, ((0, 0), (0, 0), (0, 128 - x.shape[2])))

    idx1 = _sel1(x)
    nb1 = _sc_gather(xpad.reshape(B * N, 128), idx1.reshape(-1))
    x1, idx2 = _convsel(x, nb1.reshape(B, K, N, 128), *ec['ec1'], 8, 64)
    nb2 = _sc_gather(x1.reshape(B * N, 128), idx2.reshape(-1))
    x2, idx3 = _convsel(x1, nb2.reshape(B, K, N, 128), *ec['ec2'], 64, 64)
    nb3 = _sc_gather(x2.reshape(B * N, 128), idx3.reshape(-1))
    x3, idx4 = _convsel(x2, nb3.reshape(B, K, N, 128), *ec['ec3'], 64, 128)
    nb4 = _sc_gather(x3.reshape(B * N, 128), idx4.reshape(-1))
    x_feats = _fin(x3.reshape(B, N, 128), nb4.reshape(B, K, N, 128),
                   *ec['ec4'], x1, x2, aggW, aggb)
    x_feats = x_feats.reshape(B, 128)

    hvals = [x_feats, metrics, p['tab_W0'], p['tab_b0'].reshape(1, -1)]
    for i in range(5):
        hvals += [p['tab_att_W%d' % i], p['tab_att_b%d' % i].reshape(1, -1),
                  p['tab_feat_W%d' % i], p['tab_feat_b%d' % i].reshape(1, -1)]
    for nm in ('h1', 'h2', 'h3', 'h4'):
        hvals += [p[nm + '_W'], p[nm + '_b'].reshape(1, -1)]
    hspecs = [_full(v.shape) for v in hvals]

    amax, probs, logits, mfeats = pl.pallas_call(
        _head_body,
        in_specs=hspecs,
        out_specs=[_full((B, 1)), _full((B, 50)), _full((B, 50)),
                   _full((B, 320))],
        out_shape=[
            jax.ShapeDtypeStruct((B, 1), jnp.int32),
            jax.ShapeDtypeStruct((B, 50), jnp.float32),
            jax.ShapeDtypeStruct((B, 50), jnp.float32),
            jax.ShapeDtypeStruct((B, 320), jnp.float32),
        ],
    )(*hvals)

    return (amax.reshape(B), probs, logits, x_feats, mfeats)


# speedup vs baseline: 1.5401x; 1.1333x over previous
import functools
import jax
import jax.numpy as jnp
from jax import lax
from jax.experimental import pallas as pl
from jax.experimental.pallas import tpu as pltpu
from jax.experimental.pallas import tpu_sc as plsc

K = 20
N = 1024
NEG = -3.0e38
NW = 32
CH = 128


def _leaky(z):
    return jnp.where(z > 0, z, 0.2 * z)


def _dist(X, D_ref):
    sq = jnp.sum(X * X, axis=1)
    G = lax.dot_general(X, X, (((1,), (1,)), ((), ())),
                        preferred_element_type=jnp.float32)
    D_ref[...] = 2.0 * G - sq[:, None] - sq[None, :]


def _topk_idx(D_ref, boff):
    jj = lax.broadcasted_iota(jnp.int32, (N, N), 1)
    rr = lax.broadcasted_iota(jnp.int32, (K, N), 0)

    def body(t, Imat):
        D = D_ref[...]
        m = jnp.max(D, axis=1)
        eq = D == m[:, None]
        idx = jnp.min(jnp.where(eq, jj, N + 1), axis=1)
        H = jj == idx[:, None]
        D_ref[...] = jnp.where(H, NEG, D)
        return jnp.where(rr == t, (idx + boff)[None, :], Imat)

    return lax.fori_loop(0, K, body, jnp.zeros((K, N), jnp.int32))


def _sel1_body(x_ref, idx_ref, D_ref):
    b = pl.program_id(0)
    _dist(x_ref[0], D_ref)
    idx_ref[0] = _topk_idx(D_ref, b * N)


def _conv(X, nbrs_ref, W, bvec, C, Cout):
    M = jnp.full((N, Cout), NEG, jnp.float32)
    for t in range(K):
        Sn = nbrs_ref[0, t][:, :C]
        e = jnp.concatenate([Sn - X, X], axis=1)
        h = jnp.dot(e, W, preferred_element_type=jnp.float32) + bvec
        M = jnp.maximum(M, h)
    return _leaky(M)


def _convsel_body(C, Cout, xprev_ref, nbrs_ref, W_ref, b_ref,
                  xout_ref, idx_ref, D_ref):
    b = pl.program_id(0)
    X = xprev_ref[0][:, :C]
    Xn = _conv(X, nbrs_ref, W_ref[...], b_ref[...], C, Cout)
    if Cout < 128:
        xout_ref[0] = jnp.concatenate(
            [Xn, jnp.zeros((N, 128 - Cout), jnp.float32)], axis=1)
    else:
        xout_ref[0] = Xn
    _dist(Xn, D_ref)
    idx_ref[0] = _topk_idx(D_ref, b * N)


def _fin_body(x3_ref, nbrs_ref, W_ref, b_ref, x1_ref, x2_ref,
              agg_W, agg_b, xfeat_ref):
    X3 = x3_ref[0]
    X4 = _conv(X3, nbrs_ref, W_ref[...], b_ref[...], 128, 256)
    F = jnp.concatenate([x1_ref[0][:, :64], x2_ref[0][:, :64], X3, X4],
                        axis=1)
    f = _leaky(jnp.dot(F, agg_W[...], preferred_element_type=jnp.float32)
               + agg_b[...])
    xfeat_ref[0, 0] = jnp.max(f, axis=0)


def _sc_gather(table, idx):
    M = idx.shape[0]
    Cp = table.shape[1]
    per_w = M // NW
    G = 4
    ng = per_w // (G * CH)
    mesh = plsc.VectorSubcoreMesh(core_axis_name="c", subcore_axis_name="s")

    @functools.partial(
        pl.kernel, mesh=mesh,
        out_type=jax.ShapeDtypeStruct((M, Cp), jnp.float32),
        scratch_types=[
            pltpu.VMEM((per_w,), jnp.int32),
            pltpu.VMEM((G, CH, Cp), jnp.float32),
            pltpu.SemaphoreType.DMA,
            pltpu.SemaphoreType.DMA,
        ],
    )
    def k(table_hbm, idx_hbm, out_hbm, idx_v, rows_v, semg, sems):
        wid = lax.axis_index("s") * 2 + lax.axis_index("c")
        base = wid * per_w
        pltpu.sync_copy(idx_hbm.at[pl.ds(base, per_w)], idx_v)

        def group(g, _):
            goff = g * (G * CH)
            gath = [pltpu.async_copy(
                table_hbm.at[idx_v.at[pl.ds(goff + u * CH, CH)]],
                rows_v.at[u], semg) for u in range(G)]
            for c in gath:
                c.wait()
            scat = [pltpu.async_copy(
                rows_v.at[u], out_hbm.at[pl.ds(base + goff + u * CH, CH)],
                sems) for u in range(G)]
            for c in scat:
                c.wait()
            return 0

        lax.fori_loop(0, ng, group, 0)

    return k(table, idx)


def _head_body(xfeat_ref, metrics_ref,
               tab_W0, tab_b0,
               att_W0, att_b0, feat_W0, feat_b0,
               att_W1, att_b1, feat_W1, feat_b1,
               att_W2, att_b2, feat_W2, feat_b2,
               att_W3, att_b3, feat_W3, feat_b3,
               att_W4, att_b4, feat_W4, feat_b4,
               h1_W, h1_b, h2_W, h2_b, h3_W, h3_b, h4_W, h4_b,
               amax_ref, probs_ref, logits_ref, mfeats_ref):
    metrics = metrics_ref[...]
    n_d = 64
    gamma = 1.5

    def mm(a, w, b):
        return jnp.dot(a, w[...], preferred_element_type=jnp.float32) + b[...]

    prior = jnp.ones_like(metrics)
    feat0 = jnp.maximum(mm(metrics, tab_W0, tab_b0), 0.0)
    a = feat0[:, n_d:]
    atts = ((att_W0, att_b0, feat_W0, feat_b0),
            (att_W1, att_b1, feat_W1, feat_b1),
            (att_W2, att_b2, feat_W2, feat_b2),
            (att_W3, att_b3, feat_W3, feat_b3),
            (att_W4, att_b4, feat_W4, feat_b4))
    for i, (aW, ab, fW, fb) in enumerate(atts):
        mask_logits = mm(a, aW, ab) * prior
        z = mask_logits - jnp.max(mask_logits, axis=-1, keepdims=True)
        ez = jnp.exp(z)
        mask = ez / jnp.sum(ez, axis=-1, keepdims=True)
        prior = prior * (gamma - mask)
        masked = metrics * mask
        feat = jnp.maximum(mm(masked, fW, fb), 0.0)
        mfeats_ref[:, i * n_d:(i + 1) * n_d] = feat[:, :n_d]
        a = feat[:, n_d:]

    feats = jnp.concatenate([xfeat_ref[...], mfeats_ref[...]], axis=1)
    h = jnp.maximum(mm(feats, h1_W, h1_b), 0.0)
    h = jnp.maximum(mm(h, h2_W, h2_b), 0.0)
    h = jnp.maximum(mm(h, h3_W, h3_b), 0.0)
    logits = mm(h, h4_W, h4_b)
    logits_ref[...] = logits
    z = logits - jnp.max(logits, axis=-1, keepdims=True)
    ez = jnp.exp(z)
    probs = ez / jnp.sum(ez, axis=-1, keepdims=True)
    probs_ref[...] = probs
    nc = probs.shape[1]
    ii = lax.broadcasted_iota(jnp.int32, probs.shape, 1)
    pm = jnp.max(probs, axis=1, keepdims=True)
    amax_ref[...] = jnp.min(jnp.where(probs == pm, ii, nc + 1), axis=1,
                            keepdims=True)


def _full(s):
    return pl.BlockSpec(s, lambda *_: (0,) * len(s))


def _sel1(x):
    B = x.shape[0]
    return pl.pallas_call(
        _sel1_body,
        grid=(B,),
        in_specs=[pl.BlockSpec((1, N, x.shape[2]), lambda b: (b, 0, 0))],
        out_specs=pl.BlockSpec((1, K, N), lambda b: (b, 0, 0)),
        out_shape=jax.ShapeDtypeStruct((B, K, N), jnp.int32),
        scratch_shapes=[pltpu.VMEM((N, N), jnp.float32)],
    )(x)


def _convsel(xprev, nbrs, W, bvec, C, Cout):
    B = xprev.shape[0]
    Cp = nbrs.shape[-1]
    return pl.pallas_call(
        functools.partial(_convsel_body, C, Cout),
        grid=(B,),
        in_specs=[
            pl.BlockSpec((1, N, xprev.shape[2]), lambda b: (b, 0, 0)),
            pl.BlockSpec((1, K, N, Cp), lambda b: (b, 0, 0, 0)),
            _full(W.shape), _full(bvec.shape),
        ],
        out_specs=[
            pl.BlockSpec((1, N, 128), lambda b: (b, 0, 0)),
            pl.BlockSpec((1, K, N), lambda b: (b, 0, 0)),
        ],
        out_shape=[
            jax.ShapeDtypeStruct((B, N, 128), jnp.float32),
            jax.ShapeDtypeStruct((B, K, N), jnp.int32),
        ],
        scratch_shapes=[pltpu.VMEM((N, N), jnp.float32)],
    )(xprev, nbrs, W, bvec)


def _fin(x3, nbrs, W, bvec, x1, x2, aggW, aggb):
    B = x3.shape[0]
    Cp = nbrs.shape[-1]
    return pl.pallas_call(
        _fin_body,
        grid=(B,),
        in_specs=[
            pl.BlockSpec((1, N, 128), lambda b: (b, 0, 0)),
            pl.BlockSpec((1, K, N, Cp), lambda b: (b, 0, 0, 0)),
            _full(W.shape), _full(bvec.shape),
            pl.BlockSpec((1, N, 128), lambda b: (b, 0, 0)),
            pl.BlockSpec((1, N, 128), lambda b: (b, 0, 0)),
            _full(aggW.shape), _full(aggb.shape),
        ],
        out_specs=pl.BlockSpec((1, 1, 128), lambda b: (b, 0, 0)),
        out_shape=jax.ShapeDtypeStruct((B, 1, 128), jnp.float32),
    )(x3, nbrs, W, bvec, x1, x2, aggW, aggb)


def kernel(x, metrics, params):
    p = params
    B = x.shape[0]

    ec = {nm: (p[nm + '_W'], p[nm + '_b'].reshape(1, -1))
          for nm in ('ec1', 'ec2', 'ec3', 'ec4')}
    aggW = p['agg_W']
    aggb = p['agg_b'].reshape(1, -1)

    xpad = jnp.pad(x, ((0, 0), (0, 0), (0, 128 - x.shape[2])))

    idx1 = _sel1(x)
    nb1 = _sc_gather(xpad.reshape(B * N, 128), idx1.reshape(-1))
    x1, idx2 = _convsel(x, nb1.reshape(B, K, N, 128), *ec['ec1'], 8, 64)
    nb2 = _sc_gather(x1.reshape(B * N, 128), idx2.reshape(-1))
    x2, idx3 = _convsel(x1, nb2.reshape(B, K, N, 128), *ec['ec2'], 64, 64)
    nb3 = _sc_gather(x2.reshape(B * N, 128), idx3.reshape(-1))
    x3, idx4 = _convsel(x2, nb3.reshape(B, K, N, 128), *ec['ec3'], 64, 128)
    nb4 = _sc_gather(x3.reshape(B * N, 128), idx4.reshape(-1))
    x_feats = _fin(x3.reshape(B, N, 128), nb4.reshape(B, K, N, 128),
                   *ec['ec4'], x1, x2, aggW, aggb)
    x_feats = x_feats.reshape(B, 128)

    hvals = [x_feats, metrics, p['tab_W0'], p['tab_b0'].reshape(1, -1)]
    for i in range(5):
        hvals += [p['tab_att_W%d' % i], p['tab_att_b%d' % i].reshape(1, -1),
                  p['tab_feat_W%d' % i], p['tab_feat_b%d' % i].reshape(1, -1)]
    for nm in ('h1', 'h2', 'h3', 'h4'):
        hvals += [p[nm + '_W'], p[nm + '_b'].reshape(1, -1)]
    hspecs = [_full(v.shape) for v in hvals]

    amax, probs, logits, mfeats = pl.pallas_call(
        _head_body,
        in_specs=hspecs,
        out_specs=[_full((B, 1)), _full((B, 50)), _full((B, 50)),
                   _full((B, 320))],
        out_shape=[
            jax.ShapeDtypeStruct((B, 1), jnp.int32),
            jax.ShapeDtypeStruct((B, 50), jnp.float32),
            jax.ShapeDtypeStruct((B, 50), jnp.float32),
            jax.ShapeDtypeStruct((B, 320), jnp.float32),
        ],
    )(*hvals)

    return (amax.reshape(B), probs, logits, x_feats, mfeats)


# speedup vs baseline: 1.8546x; 1.2042x over previous
import functools
import jax
import jax.numpy as jnp
from jax import lax
from jax.experimental import pallas as pl
from jax.experimental.pallas import tpu as pltpu
from jax.experimental.pallas import tpu_sc as plsc

K = 20
N = 1024
NEG = -3.0e38
NW = 32
CH = 128


def _leaky(z):
    return jnp.where(z > 0, z, 0.2 * z)


def _dist(X, D_ref):
    sq = jnp.sum(X * X, axis=1)
    G = lax.dot_general(X, X, (((1,), (1,)), ((), ())),
                        preferred_element_type=jnp.float32)
    D_ref[...] = 2.0 * G - sq[:, None] - sq[None, :]


def _topk_idx(D_ref, boff):
    jj = lax.broadcasted_iota(jnp.int32, (N, N), 1)
    rr = lax.broadcasted_iota(jnp.int32, (K, N), 0)

    def body(t, Imat):
        D = D_ref[...]
        m = jnp.max(D, axis=1)
        eq = D == m[:, None]
        idx = jnp.min(jnp.where(eq, jj, N + 1), axis=1)
        H = jj == idx[:, None]
        D_ref[...] = jnp.where(H, NEG, D)
        return jnp.where(rr == t, (idx + boff)[None, :], Imat)

    return lax.fori_loop(0, K, body, jnp.zeros((K, N), jnp.int32))


def _sel1_body(x_ref, idx_ref, D_ref):
    b = pl.program_id(0)
    _dist(x_ref[0], D_ref)
    idx_ref[0] = _topk_idx(D_ref, b * N)


def _conv(X, nbrs_ref, W, bvec, C, Cout):
    M = jnp.full((N, Cout), NEG, jnp.float32)
    for t in range(K):
        Sn = nbrs_ref[0, t][:, :C]
        e = jnp.concatenate([Sn - X, X], axis=1)
        h = jnp.dot(e, W, preferred_element_type=jnp.float32) + bvec
        M = jnp.maximum(M, h)
    return _leaky(M)


def _convsel_body(C, Cout, xprev_ref, nbrs_ref, W_ref, b_ref,
                  xout_ref, idx_ref, D_ref):
    b = pl.program_id(0)
    X = xprev_ref[0][:, :C]
    Xn = _conv(X, nbrs_ref, W_ref[...], b_ref[...], C, Cout)
    if Cout < 128:
        xout_ref[0] = jnp.concatenate(
            [Xn, jnp.zeros((N, 128 - Cout), jnp.float32)], axis=1)
    else:
        xout_ref[0] = Xn
    _dist(Xn, D_ref)
    idx_ref[0] = _topk_idx(D_ref, b * N)


def _fin_body(x3_ref, nbrs_ref, W_ref, b_ref, x1_ref, x2_ref,
              agg_W, agg_b, xfeat_ref):
    X3 = x3_ref[0]
    X4 = _conv(X3, nbrs_ref, W_ref[...], b_ref[...], 128, 256)
    F = jnp.concatenate([x1_ref[0][:, :64], x2_ref[0][:, :64], X3, X4],
                        axis=1)
    f = _leaky(jnp.dot(F, agg_W[...], preferred_element_type=jnp.float32)
               + agg_b[...])
    xfeat_ref[0, 0] = jnp.max(f, axis=0)


def _sc_gather(table, idx):
    M = idx.shape[0]
    Cp = table.shape[1]
    per_w = M // NW
    G = 6
    ng = per_w // (G * CH)
    mesh = plsc.VectorSubcoreMesh(core_axis_name="c", subcore_axis_name="s")

    @functools.partial(
        pl.kernel, mesh=mesh,
        out_type=jax.ShapeDtypeStruct((M, Cp), jnp.float32),
        scratch_types=[
            pltpu.VMEM((per_w,), jnp.int32),
            pltpu.VMEM((G, CH, Cp), jnp.float32),
            pltpu.SemaphoreType.DMA,
            pltpu.SemaphoreType.DMA,
        ],
    )
    def k(table_hbm, idx_hbm, out_hbm, idx_v, rows_v, semg, sems):
        wid = lax.axis_index("s") * 2 + lax.axis_index("c")
        base = wid * per_w
        pltpu.sync_copy(idx_hbm.at[pl.ds(base, per_w)], idx_v)

        def group(g, _):
            goff = g * (G * CH)
            gath = [pltpu.async_copy(
                table_hbm.at[idx_v.at[pl.ds(goff + u * CH, CH)]],
                rows_v.at[u], semg) for u in range(G)]
            for c in gath:
                c.wait()
            scat = [pltpu.async_copy(
                rows_v.at[u], out_hbm.at[pl.ds(base + goff + u * CH, CH)],
                sems) for u in range(G)]
            for c in scat:
                c.wait()
            return 0

        lax.fori_loop(0, ng, group, 0)

    return k(table, idx)


def _head_body(xfeat_ref, metrics_ref,
               tab_W0, tab_b0,
               att_W0, att_b0, feat_W0, feat_b0,
               att_W1, att_b1, feat_W1, feat_b1,
               att_W2, att_b2, feat_W2, feat_b2,
               att_W3, att_b3, feat_W3, feat_b3,
               att_W4, att_b4, feat_W4, feat_b4,
               h1_W, h1_b, h2_W, h2_b, h3_W, h3_b, h4_W, h4_b,
               amax_ref, probs_ref, logits_ref, mfeats_ref):
    metrics = metrics_ref[...]
    n_d = 64
    gamma = 1.5

    def mm(a, w, b):
        return jnp.dot(a, w[...], preferred_element_type=jnp.float32) + b[...]

    prior = jnp.ones_like(metrics)
    feat0 = jnp.maximum(mm(metrics, tab_W0, tab_b0), 0.0)
    a = feat0[:, n_d:]
    atts = ((att_W0, att_b0, feat_W0, feat_b0),
            (att_W1, att_b1, feat_W1, feat_b1),
            (att_W2, att_b2, feat_W2, feat_b2),
            (att_W3, att_b3, feat_W3, feat_b3),
            (att_W4, att_b4, feat_W4, feat_b4))
    for i, (aW, ab, fW, fb) in enumerate(atts):
        mask_logits = mm(a, aW, ab) * prior
        z = mask_logits - jnp.max(mask_logits, axis=-1, keepdims=True)
        ez = jnp.exp(z)
        mask = ez / jnp.sum(ez, axis=-1, keepdims=True)
        prior = prior * (gamma - mask)
        masked = metrics * mask
        feat = jnp.maximum(mm(masked, fW, fb), 0.0)
        mfeats_ref[:, i * n_d:(i + 1) * n_d] = feat[:, :n_d]
        a = feat[:, n_d:]

    feats = jnp.concatenate([xfeat_ref[...], mfeats_ref[...]], axis=1)
    h = jnp.maximum(mm(feats, h1_W, h1_b), 0.0)
    h = jnp.maximum(mm(h, h2_W, h2_b), 0.0)
    h = jnp.maximum(mm(h, h3_W, h3_b), 0.0)
    logits = mm(h, h4_W, h4_b)
    logits_ref[...] = logits
    z = logits - jnp.max(logits, axis=-1, keepdims=True)
    ez = jnp.exp(z)
    probs = ez / jnp.sum(ez, axis=-1, keepdims=True)
    probs_ref[...] = probs
    nc = probs.shape[1]
    ii = lax.broadcasted_iota(jnp.int32, probs.shape, 1)
    pm = jnp.max(probs, axis=1, keepdims=True)
    amax_ref[...] = jnp.min(jnp.where(probs == pm, ii, nc + 1), axis=1,
                            keepdims=True)


def _full(s):
    return pl.BlockSpec(s, lambda *_: (0,) * len(s))


def _sel1(x):
    B = x.shape[0]
    return pl.pallas_call(
        _sel1_body,
        grid=(B,),
        in_specs=[pl.BlockSpec((1, N, x.shape[2]), lambda b: (b, 0, 0))],
        out_specs=pl.BlockSpec((1, K, N), lambda b: (b, 0, 0)),
        out_shape=jax.ShapeDtypeStruct((B, K, N), jnp.int32),
        scratch_shapes=[pltpu.VMEM((N, N), jnp.float32)],
    )(x)


def _convsel(xprev, nbrs, W, bvec, C, Cout):
    B = xprev.shape[0]
    Cp = nbrs.shape[-1]
    return pl.pallas_call(
        functools.partial(_convsel_body, C, Cout),
        grid=(B,),
        in_specs=[
            pl.BlockSpec((1, N, xprev.shape[2]), lambda b: (b, 0, 0)),
            pl.BlockSpec((1, K, N, Cp), lambda b: (b, 0, 0, 0)),
            _full(W.shape), _full(bvec.shape),
        ],
        out_specs=[
            pl.BlockSpec((1, N, 128), lambda b: (b, 0, 0)),
            pl.BlockSpec((1, K, N), lambda b: (b, 0, 0)),
        ],
        out_shape=[
            jax.ShapeDtypeStruct((B, N, 128), jnp.float32),
            jax.ShapeDtypeStruct((B, K, N), jnp.int32),
        ],
        scratch_shapes=[pltpu.VMEM((N, N), jnp.float32)],
    )(xprev, nbrs, W, bvec)


def _fin(x3, nbrs, W, bvec, x1, x2, aggW, aggb):
    B = x3.shape[0]
    Cp = nbrs.shape[-1]
    return pl.pallas_call(
        _fin_body,
        grid=(B,),
        in_specs=[
            pl.BlockSpec((1, N, 128), lambda b: (b, 0, 0)),
            pl.BlockSpec((1, K, N, Cp), lambda b: (b, 0, 0, 0)),
            _full(W.shape), _full(bvec.shape),
            pl.BlockSpec((1, N, 128), lambda b: (b, 0, 0)),
            pl.BlockSpec((1, N, 128), lambda b: (b, 0, 0)),
            _full(aggW.shape), _full(aggb.shape),
        ],
        out_specs=pl.BlockSpec((1, 1, 128), lambda b: (b, 0, 0)),
        out_shape=jax.ShapeDtypeStruct((B, 1, 128), jnp.float32),
    )(x3, nbrs, W, bvec, x1, x2, aggW, aggb)


def kernel(x, metrics, params):
    p = params
    B = x.shape[0]

    ec = {nm: (p[nm + '_W'], p[nm + '_b'].reshape(1, -1))
          for nm in ('ec1', 'ec2', 'ec3', 'ec4')}
    aggW = p['agg_W']
    aggb = p['agg_b'].reshape(1, -1)

    xpad = jnp.pad(x, ((0, 0), (0, 0), (0, 128 - x.shape[2])))

    def dgcnn_chain(xc, xpadc):
        Bc = xc.shape[0]
        idx1 = _sel1(xc)
        nb1 = _sc_gather(xpadc.reshape(Bc * N, 128), idx1.reshape(-1))
        x1, idx2 = _convsel(xc, nb1.reshape(Bc, K, N, 128), *ec['ec1'], 8, 64)
        nb2 = _sc_gather(x1.reshape(Bc * N, 128), idx2.reshape(-1))
        x2, idx3 = _convsel(x1, nb2.reshape(Bc, K, N, 128), *ec['ec2'], 64, 64)
        nb3 = _sc_gather(x2.reshape(Bc * N, 128), idx3.reshape(-1))
        x3, idx4 = _convsel(x2, nb3.reshape(Bc, K, N, 128), *ec['ec3'], 64, 128)
        nb4 = _sc_gather(x3.reshape(Bc * N, 128), idx4.reshape(-1))
        xf = _fin(x3.reshape(Bc, N, 128), nb4.reshape(Bc, K, N, 128),
                  *ec['ec4'], x1, x2, aggW, aggb)
        return xf.reshape(Bc, 128)

    half = B // 2
    x_feats = jnp.concatenate(
        [dgcnn_chain(x[:half], xpad[:half]),
         dgcnn_chain(x[half:], xpad[half:])], axis=0)

    hvals = [x_feats, metrics, p['tab_W0'], p['tab_b0'].reshape(1, -1)]
    for i in range(5):
        hvals += [p['tab_att_W%d' % i], p['tab_att_b%d' % i].reshape(1, -1),
                  p['tab_feat_W%d' % i], p['tab_feat_b%d' % i].reshape(1, -1)]
    for nm in ('h1', 'h2', 'h3', 'h4'):
        hvals += [p[nm + '_W'], p[nm + '_b'].reshape(1, -1)]
    hspecs = [_full(v.shape) for v in hvals]

    amax, probs, logits, mfeats = pl.pallas_call(
        _head_body,
        in_specs=hspecs,
        out_specs=[_full((B, 1)), _full((B, 50)), _full((B, 50)),
                   _full((B, 320))],
        out_shape=[
            jax.ShapeDtypeStruct((B, 1), jnp.int32),
            jax.ShapeDtypeStruct((B, 50), jnp.float32),
            jax.ShapeDtypeStruct((B, 50), jnp.float32),
            jax.ShapeDtypeStruct((B, 320), jnp.float32),
        ],
    )(*hvals)

    return (amax.reshape(B), probs, logits, x_feats, mfeats)
